# baseline (speedup 1.0000x reference)
import functools

import jax
import jax.numpy as jnp
from jax import lax
from jax.experimental import pallas as pl
from jax.experimental.pallas import tpu as pltpu
from jax.experimental.pallas import tpu_sc as plsc

N = 10000
E = 320000
D = 128
B = 16

NC = 2
NS = 16
NW = NC * NS

L_CHUNK = 32
NB = 4
ZB = 32
E_TILE = -(-E // (NW * L_CHUNK * NB)) * (L_CHUNK * NB)
CH = E_TILE // L_CHUNK
GROUPS = CH // NB
E_PAD = E_TILE * NW

N_PAD = 10240
ROWS_PER_TILE = N_PAD // NS

BLK = 512
NBLK = N_PAD // BLK

assert CH % NB == 0
assert L_CHUNK >= ZB and ROWS_PER_TILE % ZB == 0


@functools.cache
def _make_edge_agg():
    mesh = plsc.VectorSubcoreMesh(core_axis_name="c", subcore_axis_name="s",
                                  num_cores=NC, num_subcores=NS)

    def body(x_hbm, src_hbm, dst_hbm, out_hbm, src_v, dst_v,
             rows0, rows1, rows2, rows3, acc,
             gsem0, gsem1, gsem2, gsem3, ssem0, ssem1, ssem2, ssem3):
        cid = lax.axis_index("c")
        sid = lax.axis_index("s")
        stripe = sid * ROWS_PER_TILE
        rows = [rows0, rows1, rows2, rows3]
        gsem = [gsem0, gsem1, gsem2, gsem3]
        ssem = [ssem0, ssem1, ssem2, ssem3]

        def zrow(i, carry):
            for k in range(D // 16):
                rows0[i, pl.ds(k * 16, 16)] = jnp.zeros((16,), jnp.float32)
            return carry
        lax.fori_loop(0, ZB, zrow, 0)
        for r in range(ROWS_PER_TILE // ZB):
            pltpu.sync_copy(rows0.at[pl.ds(0, ZB)],
                            acc.at[pl.ds(stripe + r * ZB, ZB)])

        pltpu.sync_copy(src_hbm.at[cid, sid], src_v)
        pltpu.sync_copy(dst_hbm.at[cid, sid], dst_v)
        plsc.subcore_barrier()

        def wait_sem(b, sem):
            pltpu.make_async_copy(x_hbm.at[pl.ds(0, L_CHUNK)], rows[b],
                                  sem[b]).wait()

        for b in range(NB):
            pltpu.async_copy(x_hbm.at[src_v.at[b]], rows[b], gsem[b])

        def group(i, carry):
            for b in range(NB):
                wait_sem(b, gsem)
                pltpu.async_copy(rows[b], acc.at[dst_v.at[NB * i + b]],
                                 ssem[b], add=True)

            @pl.when(i < GROUPS - 1)
            def _():
                for b in range(NB):
                    wait_sem(b, ssem)
                    pltpu.async_copy(x_hbm.at[src_v.at[NB * (i + 1) + b]],
                                     rows[b], gsem[b])
            return carry
        lax.fori_loop(0, GROUPS, group, 0)
        for b in range(NB):
            wait_sem(b, ssem)

        plsc.subcore_barrier()

        pltpu.sync_copy(acc.at[pl.ds(stripe, ROWS_PER_TILE)],
                        out_hbm.at[cid, pl.ds(stripe, ROWS_PER_TILE)])

    return pl.kernel(
        body,
        out_type=jax.ShapeDtypeStruct((NC, N_PAD, D), jnp.float32),
        mesh=mesh,
        compiler_params=pltpu.CompilerParams(use_tc_tiling_on_sc=False),
        scratch_types=(
            [pltpu.VMEM((CH, L_CHUNK), jnp.int32)] * 2
            + [pltpu.VMEM((L_CHUNK, D), jnp.float32)] * NB
            + [pltpu.VMEM_SHARED((N_PAD, D), jnp.float32)]
            + [pltpu.SemaphoreType.DMA] * (2 * NB)
        ),
    )


def _edge_agg(x_pad, srcp, dstp):
    return _make_edge_agg()(x_pad, srcp, dstp)


def _dense_body(agg_ref, x_ref, bid_ref, wa_ref, ba_ref, wb_ref, bb_ref,
                gm_ref, bt_ref, z_ref, g_ref, zp_scr):
    wa = wa_ref[...]
    wb = wb_ref[...]
    ba = ba_ref[...]
    bb = bb_ref[...]

    def pass1(i, carry):
        P, cnt, SS = carry
        a = (agg_ref[0, pl.ds(i * BLK, BLK), :] + agg_ref[1, pl.ds(i * BLK, BLK), :]
             + x_ref[pl.ds(i * BLK, BLK), :])
        h = jnp.maximum(
            lax.dot_general(a, wa, (((1,), (1,)), ((), ())),
                            preferred_element_type=jnp.float32) + ba, 0.0)
        h = lax.dot_general(h, wb, (((1,), (1,)), ((), ())),
                            preferred_element_type=jnp.float32) + bb
        zp = jnp.maximum(h, 0.0)
        bidb = bid_ref[pl.ds(i * BLK, BLK), :]
        oh = (bidb == lax.broadcasted_iota(jnp.int32, (BLK, B), 1)
              ).astype(jnp.float32)
        P = P + lax.dot_general(oh, zp, (((0,), (0,)), ((), ())),
                                preferred_element_type=jnp.float32)
        cnt = cnt + lax.dot_general(oh, jnp.ones((BLK, 1), jnp.float32),
                                    (((0,), (0,)), ((), ())),
                                    preferred_element_type=jnp.float32)
        valid = (bidb >= 0).astype(jnp.float32)
        SS = SS + jnp.sum(zp * zp * valid, axis=0, keepdims=True)
        zp_scr[pl.ds(i * BLK, BLK), :] = zp
        return P, cnt, SS

    P0 = jnp.zeros((B, D), jnp.float32)
    cnt0 = jnp.zeros((B, 1), jnp.float32)
    SS0 = jnp.zeros((1, D), jnp.float32)
    P, cnt, SS = lax.fori_loop(0, NBLK, pass1, (P0, cnt0, SS0))

    mean = jnp.sum(P, axis=0, keepdims=True) / N
    var = SS / N - mean * mean
    inv = lax.rsqrt(var + 1e-5)
    k = gm_ref[...] * inv
    c = bt_ref[...] - mean * k
    g_ref[...] = P * k + cnt * c

    def pass2(i, carry):
        z_ref[pl.ds(i * BLK, BLK), :] = zp_scr[pl.ds(i * BLK, BLK), :] * k + c
        return carry
    lax.fori_loop(0, NBLK, pass2, 0)


def _dense(agg, xin, bid, Wa, ba, Wb, bb, gamma, beta):
    return pl.pallas_call(
        _dense_body,
        out_shape=[jax.ShapeDtypeStruct((N_PAD, D), jnp.float32),
                   jax.ShapeDtypeStruct((B, D), jnp.float32)],
        scratch_shapes=[pltpu.VMEM((N_PAD, D), jnp.float32)],
    )(agg, xin, bid, Wa, ba.reshape(1, D), Wb, bb.reshape(1, D),
      gamma.reshape(1, D), beta.reshape(1, D))


def kernel(x, edge_index, node_batch_id,
           W1a, b1a, W1b, b1b, gamma1, beta1,
           W2a, b2a, W2b, b2b, gamma2, beta2):
    src = edge_index[0]
    dst = edge_index[1]
    srcp = jnp.concatenate(
        [src, jnp.zeros((E_PAD - E,), jnp.int32)]).reshape(NC, NS, CH, L_CHUNK)
    dstp = jnp.concatenate(
        [dst, jnp.full((E_PAD - E,), N, jnp.int32)]).reshape(NC, NS, CH, L_CHUNK)
    bidp = jnp.concatenate(
        [node_batch_id, jnp.full((N_PAD - N,), -1, jnp.int32)])[:, None]
    x_pad = jnp.pad(x, ((0, N_PAD - N), (0, 0)))

    agg1 = _edge_agg(x_pad, srcp, dstp)
    z1p, g1 = _dense(agg1, x_pad, bidp, W1a, b1a, W1b, b1b, gamma1, beta1)
    agg2 = _edge_agg(z1p, srcp, dstp)
    z2p, g2 = _dense(agg2, z1p, bidp, W2a, b2a, W2b, b2b, gamma2, beta2)

    z_out = jnp.concatenate([z1p[:N], z2p[:N]], axis=1)
    g_out = jnp.concatenate([g1, g2], axis=1)
    return (z_out, g_out)


# speedup vs baseline: 1.3737x; 1.3737x over previous
import functools

import jax
import jax.numpy as jnp
from jax import lax
from jax.experimental import pallas as pl
from jax.experimental.pallas import tpu as pltpu
from jax.experimental.pallas import tpu_sc as plsc

N = 10000
E = 320000
D = 128
B = 16

NC = 2
NS = 16
NW = NC * NS

L_CHUNK = 32
ZB = 32
E_TILE = -(-E // (NW * L_CHUNK)) * L_CHUNK
CH = E_TILE // L_CHUNK
E_PAD = E_TILE * NW

N_PAD = 10240
ROWS_PER_TILE = N_PAD // NS

BLK = 512
NBLK = N_PAD // BLK

assert CH % 2 == 1
assert L_CHUNK >= ZB and ROWS_PER_TILE % ZB == 0


@functools.cache
def _make_edge_agg():
    mesh = plsc.VectorSubcoreMesh(core_axis_name="c", subcore_axis_name="s",
                                  num_cores=NC, num_subcores=NS)

    def body(x_hbm, src_hbm, dst_hbm, out_hbm, src_v, dst_v, rows0, rows1,
             acc, sem0, sem1):
        cid = lax.axis_index("c")
        sid = lax.axis_index("s")
        stripe = sid * ROWS_PER_TILE

        def zrow(i, carry):
            for k in range(D // 16):
                rows0[i, pl.ds(k * 16, 16)] = jnp.zeros((16,), jnp.float32)
            return carry
        lax.fori_loop(0, ZB, zrow, 0)
        for r in range(ROWS_PER_TILE // ZB):
            pltpu.sync_copy(rows0.at[pl.ds(0, ZB)],
                            acc.at[pl.ds(stripe + r * ZB, ZB)])

        pltpu.sync_copy(src_hbm.at[cid, sid], src_v)
        pltpu.sync_copy(dst_hbm.at[cid, sid], dst_v)
        plsc.subcore_barrier()

        def wait0():
            pltpu.make_async_copy(x_hbm.at[pl.ds(0, L_CHUNK)], rows0, sem0).wait()

        def wait1():
            pltpu.make_async_copy(x_hbm.at[pl.ds(0, L_CHUNK)], rows1, sem1).wait()

        pltpu.async_copy(x_hbm.at[src_v.at[0]], rows0, sem0)

        def step(t, carry):
            pltpu.async_copy(x_hbm.at[src_v.at[2 * t + 1]], rows1, sem1)
            wait0()
            pltpu.sync_copy(rows0, acc.at[dst_v.at[2 * t]], add=True)
            pltpu.async_copy(x_hbm.at[src_v.at[2 * t + 2]], rows0, sem0)
            wait1()
            pltpu.sync_copy(rows1, acc.at[dst_v.at[2 * t + 1]], add=True)
            return carry
        lax.fori_loop(0, (CH - 1) // 2, step, 0)
        wait0()
        pltpu.sync_copy(rows0, acc.at[dst_v.at[CH - 1]], add=True)

        plsc.subcore_barrier()

        pltpu.sync_copy(acc.at[pl.ds(stripe, ROWS_PER_TILE)],
                        out_hbm.at[cid, pl.ds(stripe, ROWS_PER_TILE)])

    return pl.kernel(
        body,
        out_type=jax.ShapeDtypeStruct((NC, N_PAD, D), jnp.float32),
        mesh=mesh,
        compiler_params=pltpu.CompilerParams(use_tc_tiling_on_sc=False),
        scratch_types=[
            pltpu.VMEM((CH, L_CHUNK), jnp.int32),
            pltpu.VMEM((CH, L_CHUNK), jnp.int32),
            pltpu.VMEM((L_CHUNK, D), jnp.float32),
            pltpu.VMEM((L_CHUNK, D), jnp.float32),
            pltpu.VMEM_SHARED((N_PAD, D), jnp.float32),
            pltpu.SemaphoreType.DMA,
            pltpu.SemaphoreType.DMA,
        ],
    )


def _edge_agg(x_pad, srcp, dstp):
    return _make_edge_agg()(x_pad, srcp, dstp)


def _dense_body(agg_ref, x_ref, bid_ref, wa_ref, ba_ref, wb_ref, bb_ref,
                gm_ref, bt_ref, z_ref, g_ref, zp_scr):
    wa = wa_ref[...]
    wb = wb_ref[...]
    ba = ba_ref[...]
    bb = bb_ref[...]

    def pass1(i, carry):
        P, cnt, SS = carry
        a = (agg_ref[0, pl.ds(i * BLK, BLK), :] + agg_ref[1, pl.ds(i * BLK, BLK), :]
             + x_ref[pl.ds(i * BLK, BLK), :])
        h = jnp.maximum(
            lax.dot_general(a, wa, (((1,), (1,)), ((), ())),
                            preferred_element_type=jnp.float32) + ba, 0.0)
        h = lax.dot_general(h, wb, (((1,), (1,)), ((), ())),
                            preferred_element_type=jnp.float32) + bb
        zp = jnp.maximum(h, 0.0)
        bidb = bid_ref[pl.ds(i * BLK, BLK), :]
        oh = (bidb == lax.broadcasted_iota(jnp.int32, (BLK, B), 1)
              ).astype(jnp.float32)
        P = P + lax.dot_general(oh, zp, (((0,), (0,)), ((), ())),
                                preferred_element_type=jnp.float32)
        cnt = cnt + lax.dot_general(oh, jnp.ones((BLK, 1), jnp.float32),
                                    (((0,), (0,)), ((), ())),
                                    preferred_element_type=jnp.float32)
        valid = (bidb >= 0).astype(jnp.float32)
        SS = SS + jnp.sum(zp * zp * valid, axis=0, keepdims=True)
        zp_scr[pl.ds(i * BLK, BLK), :] = zp
        return P, cnt, SS

    P0 = jnp.zeros((B, D), jnp.float32)
    cnt0 = jnp.zeros((B, 1), jnp.float32)
    SS0 = jnp.zeros((1, D), jnp.float32)
    P, cnt, SS = lax.fori_loop(0, NBLK, pass1, (P0, cnt0, SS0))

    mean = jnp.sum(P, axis=0, keepdims=True) / N
    var = SS / N - mean * mean
    inv = lax.rsqrt(var + 1e-5)
    k = gm_ref[...] * inv
    c = bt_ref[...] - mean * k
    g_ref[...] = P * k + cnt * c

    def pass2(i, carry):
        z_ref[pl.ds(i * BLK, BLK), :] = zp_scr[pl.ds(i * BLK, BLK), :] * k + c
        return carry
    lax.fori_loop(0, NBLK, pass2, 0)


def _dense(agg, xin, bid, Wa, ba, Wb, bb, gamma, beta):
    return pl.pallas_call(
        _dense_body,
        out_shape=[jax.ShapeDtypeStruct((N_PAD, D), jnp.float32),
                   jax.ShapeDtypeStruct((B, D), jnp.float32)],
        scratch_shapes=[pltpu.VMEM((N_PAD, D), jnp.float32)],
    )(agg, xin, bid, Wa, ba.reshape(1, D), Wb, bb.reshape(1, D),
      gamma.reshape(1, D), beta.reshape(1, D))


def kernel(x, edge_index, node_batch_id,
           W1a, b1a, W1b, b1b, gamma1, beta1,
           W2a, b2a, W2b, b2b, gamma2, beta2):
    src = edge_index[0]
    dst = edge_index[1]
    srcp = jnp.concatenate(
        [src, jnp.zeros((E_PAD - E,), jnp.int32)]).reshape(NC, NS, CH, L_CHUNK)
    dstp = jnp.concatenate(
        [dst, jnp.full((E_PAD - E,), N, jnp.int32)]).reshape(NC, NS, CH, L_CHUNK)
    bidp = jnp.concatenate(
        [node_batch_id, jnp.full((N_PAD - N,), -1, jnp.int32)])[:, None]
    x_pad = jnp.pad(x, ((0, N_PAD - N), (0, 0)))

    agg1 = _edge_agg(x_pad, srcp, dstp)
    z1p, g1 = _dense(agg1, x_pad, bidp, W1a, b1a, W1b, b1b, gamma1, beta1)
    agg2 = _edge_agg(z1p, srcp, dstp)
    z2p, g2 = _dense(agg2, z1p, bidp, W2a, b2a, W2b, b2b, gamma2, beta2)

    z_out = jnp.concatenate([z1p[:N], z2p[:N]], axis=1)
    g_out = jnp.concatenate([g1, g2], axis=1)
    return (z_out, g_out)


# speedup vs baseline: 1.4012x; 1.0200x over previous
import functools

import jax
import jax.numpy as jnp
from jax import lax
from jax.experimental import pallas as pl
from jax.experimental.pallas import tpu as pltpu
from jax.experimental.pallas import tpu_sc as plsc

N = 10000
E = 320000
D = 128
B = 16

NC = 2
NS = 16
NW = NC * NS

L_CHUNK = 32
ZB = 32
E_TILE = -(-E // (NW * L_CHUNK)) * L_CHUNK
CH = E_TILE // L_CHUNK
E_PAD = E_TILE * NW

N_PAD = 10240
ROWS_PER_TILE = N_PAD // NS

BLK = 2048
NBLK = N_PAD // BLK

assert CH % 2 == 1
assert L_CHUNK >= ZB and ROWS_PER_TILE % ZB == 0


@functools.cache
def _make_edge_agg():
    mesh = plsc.VectorSubcoreMesh(core_axis_name="c", subcore_axis_name="s",
                                  num_cores=NC, num_subcores=NS)

    def body(x_hbm, src_hbm, dst_hbm, out_hbm, src_v, dst_v, rows0, rows1,
             acc, sem0, sem1):
        cid = lax.axis_index("c")
        sid = lax.axis_index("s")
        stripe = sid * ROWS_PER_TILE

        def zrow(i, carry):
            for k in range(D // 16):
                rows0[i, pl.ds(k * 16, 16)] = jnp.zeros((16,), jnp.float32)
            return carry
        lax.fori_loop(0, ZB, zrow, 0)
        for r in range(ROWS_PER_TILE // ZB):
            pltpu.sync_copy(rows0.at[pl.ds(0, ZB)],
                            acc.at[pl.ds(stripe + r * ZB, ZB)])

        pltpu.sync_copy(src_hbm.at[cid, sid], src_v)
        pltpu.sync_copy(dst_hbm.at[cid, sid], dst_v)
        plsc.subcore_barrier()

        def wait0():
            pltpu.make_async_copy(x_hbm.at[pl.ds(0, L_CHUNK)], rows0, sem0).wait()

        def wait1():
            pltpu.make_async_copy(x_hbm.at[pl.ds(0, L_CHUNK)], rows1, sem1).wait()

        pltpu.async_copy(x_hbm.at[src_v.at[0]], rows0, sem0)

        def step(t, carry):
            pltpu.async_copy(x_hbm.at[src_v.at[2 * t + 1]], rows1, sem1)
            wait0()
            pltpu.sync_copy(rows0, acc.at[dst_v.at[2 * t]], add=True)
            pltpu.async_copy(x_hbm.at[src_v.at[2 * t + 2]], rows0, sem0)
            wait1()
            pltpu.sync_copy(rows1, acc.at[dst_v.at[2 * t + 1]], add=True)
            return carry
        lax.fori_loop(0, (CH - 1) // 2, step, 0)
        wait0()
        pltpu.sync_copy(rows0, acc.at[dst_v.at[CH - 1]], add=True)

        plsc.subcore_barrier()

        pltpu.sync_copy(acc.at[pl.ds(stripe, ROWS_PER_TILE)],
                        out_hbm.at[cid, pl.ds(stripe, ROWS_PER_TILE)])

    return pl.kernel(
        body,
        out_type=jax.ShapeDtypeStruct((NC, N_PAD, D), jnp.float32),
        mesh=mesh,
        compiler_params=pltpu.CompilerParams(use_tc_tiling_on_sc=False),
        scratch_types=[
            pltpu.VMEM((CH, L_CHUNK), jnp.int32),
            pltpu.VMEM((CH, L_CHUNK), jnp.int32),
            pltpu.VMEM((L_CHUNK, D), jnp.float32),
            pltpu.VMEM((L_CHUNK, D), jnp.float32),
            pltpu.VMEM_SHARED((N_PAD, D), jnp.float32),
            pltpu.SemaphoreType.DMA,
            pltpu.SemaphoreType.DMA,
        ],
    )


def _edge_agg(x_pad, srcp, dstp):
    return _make_edge_agg()(x_pad, srcp, dstp)


def _dense_body(agg_ref, x_ref, bid_ref, wa_ref, ba_ref, wb_ref, bb_ref,
                gm_ref, bt_ref, z_ref, g_ref, zp_scr):
    wa = wa_ref[...]
    wb = wb_ref[...]
    ba = ba_ref[...]
    bb = bb_ref[...]

    def pass1(i, carry):
        P, cnt, SS = carry
        a = (agg_ref[0, pl.ds(i * BLK, BLK), :] + agg_ref[1, pl.ds(i * BLK, BLK), :]
             + x_ref[pl.ds(i * BLK, BLK), :])
        h = jnp.maximum(
            lax.dot_general(a, wa, (((1,), (1,)), ((), ())),
                            preferred_element_type=jnp.float32) + ba, 0.0)
        h = lax.dot_general(h, wb, (((1,), (1,)), ((), ())),
                            preferred_element_type=jnp.float32) + bb
        zp = jnp.maximum(h, 0.0)
        bidb = bid_ref[pl.ds(i * BLK, BLK), :]
        oh = (bidb == lax.broadcasted_iota(jnp.int32, (BLK, B), 1)
              ).astype(jnp.float32)
        P = P + lax.dot_general(oh, zp, (((0,), (0,)), ((), ())),
                                preferred_element_type=jnp.float32)
        cnt = cnt + lax.dot_general(oh, jnp.ones((BLK, 1), jnp.float32),
                                    (((0,), (0,)), ((), ())),
                                    preferred_element_type=jnp.float32)
        valid = (bidb >= 0).astype(jnp.float32)
        SS = SS + jnp.sum(zp * zp * valid, axis=0, keepdims=True)
        zp_scr[pl.ds(i * BLK, BLK), :] = zp
        return P, cnt, SS

    P0 = jnp.zeros((B, D), jnp.float32)
    cnt0 = jnp.zeros((B, 1), jnp.float32)
    SS0 = jnp.zeros((1, D), jnp.float32)
    P, cnt, SS = lax.fori_loop(0, NBLK, pass1, (P0, cnt0, SS0))

    mean = jnp.sum(P, axis=0, keepdims=True) / N
    var = SS / N - mean * mean
    inv = lax.rsqrt(var + 1e-5)
    k = gm_ref[...] * inv
    c = bt_ref[...] - mean * k
    g_ref[...] = P * k + cnt * c

    def pass2(i, carry):
        z_ref[pl.ds(i * BLK, BLK), :] = zp_scr[pl.ds(i * BLK, BLK), :] * k + c
        return carry
    lax.fori_loop(0, NBLK, pass2, 0)


def _dense(agg, xin, bid, Wa, ba, Wb, bb, gamma, beta):
    return pl.pallas_call(
        _dense_body,
        out_shape=[jax.ShapeDtypeStruct((N_PAD, D), jnp.float32),
                   jax.ShapeDtypeStruct((B, D), jnp.float32)],
        scratch_shapes=[pltpu.VMEM((N_PAD, D), jnp.float32)],
    )(agg, xin, bid, Wa, ba.reshape(1, D), Wb, bb.reshape(1, D),
      gamma.reshape(1, D), beta.reshape(1, D))


def kernel(x, edge_index, node_batch_id,
           W1a, b1a, W1b, b1b, gamma1, beta1,
           W2a, b2a, W2b, b2b, gamma2, beta2):
    src = edge_index[0]
    dst = edge_index[1]
    srcp = jnp.concatenate(
        [src, jnp.zeros((E_PAD - E,), jnp.int32)]).reshape(NC, NS, CH, L_CHUNK)
    dstp = jnp.concatenate(
        [dst, jnp.full((E_PAD - E,), N, jnp.int32)]).reshape(NC, NS, CH, L_CHUNK)
    bidp = jnp.concatenate(
        [node_batch_id, jnp.full((N_PAD - N,), -1, jnp.int32)])[:, None]
    x_pad = jnp.pad(x, ((0, N_PAD - N), (0, 0)))

    agg1 = _edge_agg(x_pad, srcp, dstp)
    z1p, g1 = _dense(agg1, x_pad, bidp, W1a, b1a, W1b, b1b, gamma1, beta1)
    agg2 = _edge_agg(z1p, srcp, dstp)
    z2p, g2 = _dense(agg2, z1p, bidp, W2a, b2a, W2b, b2b, gamma2, beta2)

    z_out = jnp.concatenate([z1p[:N], z2p[:N]], axis=1)
    g_out = jnp.concatenate([g1, g2], axis=1)
    return (z_out, g_out)


# speedup vs baseline: 1.4067x; 1.0039x over previous
import functools

import jax
import jax.numpy as jnp
from jax import lax
from jax.experimental import pallas as pl
from jax.experimental.pallas import tpu as pltpu
from jax.experimental.pallas import tpu_sc as plsc

N = 10000
E = 320000
D = 128
B = 16

NC = 2
NS = 16
NW = NC * NS

L_CHUNK = 32
ZB = 32
E_TILE = -(-E // (NW * L_CHUNK)) * L_CHUNK
CH = E_TILE // L_CHUNK
E_PAD = E_TILE * NW

N_PAD = 10240
ROWS_PER_TILE = N_PAD // NS

BLK = 10240
NBLK = N_PAD // BLK

assert CH % 2 == 1
assert L_CHUNK >= ZB and ROWS_PER_TILE % ZB == 0


@functools.cache
def _make_edge_agg():
    mesh = plsc.VectorSubcoreMesh(core_axis_name="c", subcore_axis_name="s",
                                  num_cores=NC, num_subcores=NS)

    def body(x_hbm, src_hbm, dst_hbm, out_hbm, src_v, dst_v, rows0, rows1,
             acc, sem0, sem1):
        cid = lax.axis_index("c")
        sid = lax.axis_index("s")
        stripe = sid * ROWS_PER_TILE

        def zrow(i, carry):
            for k in range(D // 16):
                rows0[i, pl.ds(k * 16, 16)] = jnp.zeros((16,), jnp.float32)
            return carry
        lax.fori_loop(0, ZB, zrow, 0)
        for r in range(ROWS_PER_TILE // ZB):
            pltpu.sync_copy(rows0.at[pl.ds(0, ZB)],
                            acc.at[pl.ds(stripe + r * ZB, ZB)])

        pltpu.sync_copy(src_hbm.at[cid, sid], src_v)
        pltpu.sync_copy(dst_hbm.at[cid, sid], dst_v)
        plsc.subcore_barrier()

        def wait0():
            pltpu.make_async_copy(x_hbm.at[pl.ds(0, L_CHUNK)], rows0, sem0).wait()

        def wait1():
            pltpu.make_async_copy(x_hbm.at[pl.ds(0, L_CHUNK)], rows1, sem1).wait()

        pltpu.async_copy(x_hbm.at[src_v.at[0]], rows0, sem0)

        def step(t, carry):
            pltpu.async_copy(x_hbm.at[src_v.at[2 * t + 1]], rows1, sem1)
            wait0()
            pltpu.sync_copy(rows0, acc.at[dst_v.at[2 * t]], add=True)
            pltpu.async_copy(x_hbm.at[src_v.at[2 * t + 2]], rows0, sem0)
            wait1()
            pltpu.sync_copy(rows1, acc.at[dst_v.at[2 * t + 1]], add=True)
            return carry
        lax.fori_loop(0, (CH - 1) // 2, step, 0)
        wait0()
        pltpu.sync_copy(rows0, acc.at[dst_v.at[CH - 1]], add=True)

        plsc.subcore_barrier()

        pltpu.sync_copy(acc.at[pl.ds(stripe, ROWS_PER_TILE)],
                        out_hbm.at[cid, pl.ds(stripe, ROWS_PER_TILE)])

    return pl.kernel(
        body,
        out_type=jax.ShapeDtypeStruct((NC, N_PAD, D), jnp.float32),
        mesh=mesh,
        compiler_params=pltpu.CompilerParams(use_tc_tiling_on_sc=False),
        scratch_types=[
            pltpu.VMEM((CH, L_CHUNK), jnp.int32),
            pltpu.VMEM((CH, L_CHUNK), jnp.int32),
            pltpu.VMEM((L_CHUNK, D), jnp.float32),
            pltpu.VMEM((L_CHUNK, D), jnp.float32),
            pltpu.VMEM_SHARED((N_PAD, D), jnp.float32),
            pltpu.SemaphoreType.DMA,
            pltpu.SemaphoreType.DMA,
        ],
    )


def _edge_agg(x_pad, srcp, dstp):
    return _make_edge_agg()(x_pad, srcp, dstp)


def _dense_body(agg_ref, x_ref, bid_ref, wa_ref, ba_ref, wb_ref, bb_ref,
                gm_ref, bt_ref, z_ref, g_ref, zp_scr):
    wa = wa_ref[...]
    wb = wb_ref[...]
    ba = ba_ref[...]
    bb = bb_ref[...]

    def pass1(i, carry):
        P, cnt, SS = carry
        a = (agg_ref[0, pl.ds(i * BLK, BLK), :] + agg_ref[1, pl.ds(i * BLK, BLK), :]
             + x_ref[pl.ds(i * BLK, BLK), :])
        h = jnp.maximum(
            lax.dot_general(a, wa, (((1,), (1,)), ((), ())),
                            preferred_element_type=jnp.float32) + ba, 0.0)
        h = lax.dot_general(h, wb, (((1,), (1,)), ((), ())),
                            preferred_element_type=jnp.float32) + bb
        zp = jnp.maximum(h, 0.0)
        bidb = bid_ref[pl.ds(i * BLK, BLK), :]
        oh = (bidb == lax.broadcasted_iota(jnp.int32, (BLK, B), 1)
              ).astype(jnp.float32)
        P = P + lax.dot_general(oh, zp, (((0,), (0,)), ((), ())),
                                preferred_element_type=jnp.float32)
        cnt = cnt + lax.dot_general(oh, jnp.ones((BLK, 1), jnp.float32),
                                    (((0,), (0,)), ((), ())),
                                    preferred_element_type=jnp.float32)
        valid = (bidb >= 0).astype(jnp.float32)
        SS = SS + jnp.sum(zp * zp * valid, axis=0, keepdims=True)
        zp_scr[pl.ds(i * BLK, BLK), :] = zp
        return P, cnt, SS

    P0 = jnp.zeros((B, D), jnp.float32)
    cnt0 = jnp.zeros((B, 1), jnp.float32)
    SS0 = jnp.zeros((1, D), jnp.float32)
    P, cnt, SS = lax.fori_loop(0, NBLK, pass1, (P0, cnt0, SS0))

    mean = jnp.sum(P, axis=0, keepdims=True) / N
    var = SS / N - mean * mean
    inv = lax.rsqrt(var + 1e-5)
    k = gm_ref[...] * inv
    c = bt_ref[...] - mean * k
    g_ref[...] = P * k + cnt * c

    def pass2(i, carry):
        z_ref[pl.ds(i * BLK, BLK), :] = zp_scr[pl.ds(i * BLK, BLK), :] * k + c
        return carry
    lax.fori_loop(0, NBLK, pass2, 0)


def _dense(agg, xin, bid, Wa, ba, Wb, bb, gamma, beta):
    return pl.pallas_call(
        _dense_body,
        out_shape=[jax.ShapeDtypeStruct((N_PAD, D), jnp.float32),
                   jax.ShapeDtypeStruct((B, D), jnp.float32)],
        scratch_shapes=[pltpu.VMEM((N_PAD, D), jnp.float32)],
    )(agg, xin, bid, Wa, ba.reshape(1, D), Wb, bb.reshape(1, D),
      gamma.reshape(1, D), beta.reshape(1, D))


def kernel(x, edge_index, node_batch_id,
           W1a, b1a, W1b, b1b, gamma1, beta1,
           W2a, b2a, W2b, b2b, gamma2, beta2):
    src = edge_index[0]
    dst = edge_index[1]
    srcp = jnp.concatenate(
        [src, jnp.zeros((E_PAD - E,), jnp.int32)]).reshape(NC, NS, CH, L_CHUNK)
    dstp = jnp.concatenate(
        [dst, jnp.full((E_PAD - E,), N, jnp.int32)]).reshape(NC, NS, CH, L_CHUNK)
    bidp = jnp.concatenate(
        [node_batch_id, jnp.full((N_PAD - N,), -1, jnp.int32)])[:, None]
    x_pad = jnp.pad(x, ((0, N_PAD - N), (0, 0)))

    agg1 = _edge_agg(x_pad, srcp, dstp)
    z1p, g1 = _dense(agg1, x_pad, bidp, W1a, b1a, W1b, b1b, gamma1, beta1)
    agg2 = _edge_agg(z1p, srcp, dstp)
    z2p, g2 = _dense(agg2, z1p, bidp, W2a, b2a, W2b, b2b, gamma2, beta2)

    z_out = jnp.concatenate([z1p[:N], z2p[:N]], axis=1)
    g_out = jnp.concatenate([g1, g2], axis=1)
    return (z_out, g_out)
